# Initial kernel scaffold; baseline (speedup 1.0000x reference)
import jax
import jax.numpy as jnp
from jax.experimental import pallas as pl


def kernel(input, target):
    raise NotImplementedError("write your pallas kernel here")


# speedup vs baseline: 4.1587x; 4.1587x over previous
import jax
import jax.numpy as jnp
from jax.experimental import pallas as pl
from jax.experimental.pallas import tpu as pltpu

_B = 8
_C = 19
_N = 512 * 512
_K = _N // 2
_CHUNK = 8192
_NCHUNK = _N // _CHUNK

def _ce_topk_kernel(x_ref, t_ref, out_ref, keys_ref):
    _SIGN = jnp.uint32(0x80000000)
    _MASK = jnp.uint32(0x7FFFFFFF)
    b = pl.program_id(0)
    c = pl.program_id(1)

    x = x_ref[0]
    t = t_ref[0]

    m = jnp.max(x, axis=0, keepdims=True)
    lse = m + jnp.log(jnp.sum(jnp.exp(x - m), axis=0, keepdims=True))
    cls = jax.lax.broadcasted_iota(jnp.int32, x.shape, 0)
    xt = jnp.sum(jnp.where(cls == t, x, 0.0), axis=0, keepdims=True)
    nll = lse - xt

    bu = jax.lax.bitcast_convert_type(nll, jnp.uint32)
    key = jnp.where(nll < 0.0, ~bu, bu | _SIGN)
    keys_ref[b, pl.ds(c * _CHUNK, _CHUNK)] = key[0]

    @pl.when((b == _B - 1) & (c == _NCHUNK - 1))
    def _select():
        keys = keys_ref[...]

        def body(i, thr):
            bit = (31 - i).astype(jnp.uint32)
            cand = thr | jax.lax.shift_left(jnp.uint32(1), bit)
            cnt = jnp.sum((keys >= cand).astype(jnp.int32), axis=1,
                          keepdims=True)
            return jnp.where(cnt >= _K, cand, thr)

        thr = jax.lax.fori_loop(0, 32, body, jnp.zeros((_B, 1), jnp.uint32))

        vbits = jnp.where(keys >= _SIGN, keys & _MASK, ~keys)
        vals = jax.lax.bitcast_convert_type(vbits, jnp.float32)
        gt = keys > thr
        cnt_gt = jnp.sum(gt.astype(jnp.int32), axis=1, keepdims=True)
        sum_gt = jnp.sum(jnp.where(gt, vals, 0.0), axis=1, keepdims=True)

        tbits = jnp.where(thr >= _SIGN, thr & _MASK, ~thr)
        tval = jax.lax.bitcast_convert_type(tbits, jnp.float32)
        row = sum_gt + (_K - cnt_gt).astype(jnp.float32) * tval
        out_ref[...] = jnp.sum(row, axis=0, keepdims=True) / jnp.float32(_B * _K)


def kernel(input, target):
    x = input.reshape(_B, _C, _N)
    t = target.astype(jnp.int32).reshape(_B, 1, _N)
    out = pl.pallas_call(
        _ce_topk_kernel,
        grid=(_B, _NCHUNK),
        in_specs=[
            pl.BlockSpec((1, _C, _CHUNK), lambda b, c: (b, 0, c)),
            pl.BlockSpec((1, 1, _CHUNK), lambda b, c: (b, 0, c)),
        ],
        out_specs=pl.BlockSpec((1, 1), lambda b, c: (0, 0)),
        out_shape=jax.ShapeDtypeStruct((1, 1), jnp.float32),
        scratch_shapes=[pltpu.VMEM((_B, _N), jnp.uint32)],
    )(x, t)
    return out.reshape(())
